# baseline (speedup 1.0000x reference)
import functools

import jax
import jax.numpy as jnp
from jax import lax
from jax.experimental import pallas as pl
from jax.experimental.pallas import tpu as pltpu
from jax.experimental.pallas import tpu_sc as plsc

_LAMBDA_DIS = 1e-05
_N_NODE = 100000
_B = 16384
_D = 64
_NW = 32
_BPW = _B // _NW
_NCH = 4
_CH = _BPW // _NCH


def _sc_scores(nid, nbr, emd2, bias):
    mesh = plsc.VectorSubcoreMesh(core_axis_name="c", subcore_axis_name="s")

    @functools.partial(
        pl.kernel,
        out_type=(
            jax.ShapeDtypeStruct((_B,), jnp.float32),
            jax.ShapeDtypeStruct((_NW * 3 * 16,), jnp.float32),
        ),
        mesh=mesh,
        compiler_params=pltpu.CompilerParams(
            needs_layout_passes=False, use_tc_tiling_on_sc=True),
        scratch_types=[
            pltpu.VMEM((_NCH, _CH), jnp.int32),
            pltpu.VMEM((_NCH, _CH), jnp.int32),
            pltpu.VMEM((_NCH, _CH), jnp.int32),
            pltpu.VMEM((_NCH, _CH), jnp.int32),
            pltpu.VMEM((2, _CH, 2 * _D), jnp.float32),
            pltpu.VMEM((2, _CH, 2 * _D), jnp.float32),
            pltpu.VMEM((_BPW,), jnp.float32),
            pltpu.VMEM((_BPW,), jnp.float32),
            pltpu.VMEM((48,), jnp.float32),
            pltpu.SemaphoreType.DMA,
            pltpu.SemaphoreType.DMA,
            pltpu.SemaphoreType.DMA,
        ],
    )
    def body(nid_hbm, nbr_hbm, emd_hbm, bias_hbm, score_out, sq_out,
             idx1, idx2, row1, row2,
             rows1, rows2, biasv, scores, sqst, sem0, sem1, semb):
        wid = lax.axis_index("s") * 2 + lax.axis_index("c")
        base = wid * _BPW

        for c in range(_NCH):
            sl = pl.ds(base + c * _CH, _CH)
            pltpu.sync_copy(nid_hbm.at[sl], idx1.at[c])
            pltpu.sync_copy(nbr_hbm.at[sl], idx2.at[c])

        one = jnp.full((16,), 1, jnp.int32)
        for c in range(_NCH):
            for k in range(_CH // 16):
                sl = pl.ds(k * 16, 16)
                row1[c, sl] = lax.shift_right_logical(idx1[c, sl], one)
                row2[c, sl] = lax.shift_right_logical(idx2[c, sl], one)

        bias_cps = [
            pltpu.async_copy(bias_hbm.at[idx2.at[c]],
                             biasv.at[pl.ds(c * _CH, _CH)], semb)
            for c in range(_NCH)
        ]

        sems = (sem0, sem1)

        def start_gather(c):
            s = sems[c % 2]
            return (
                pltpu.async_copy(emd_hbm.at[row1.at[c]], rows1.at[c % 2], s),
                pltpu.async_copy(emd_hbm.at[row2.at[c]], rows2.at[c % 2], s),
            )

        lanes = lax.iota(jnp.int32, 16)
        zero = jnp.zeros((16,), jnp.float32)

        pend = start_gather(0)
        for cp in bias_cps:
            cp.wait()
        carry = (zero, zero, zero)
        for c in range(_NCH):
            nxt = start_gather(c + 1) if c + 1 < _NCH else None
            pend[0].wait()
            pend[1].wait()
            pend = nxt
            r1 = rows1.at[c % 2]
            r2 = rows2.at[c % 2]

            def group(g, carry, c=c, r1=r1, r2=r2):
                acc1, acc2, accb = carry
                gbase = pl.multiple_of(g * 16, 16)
                p1v = idx1[c, pl.ds(gbase, 16)] & one
                p2v = idx2[c, pl.ds(gbase, 16)] & one
                acc_s = zero
                for r in range(16):
                    k = gbase + r
                    rsel = jnp.full((16,), r, jnp.int32)
                    m1 = p1v[rsel] == one
                    m2 = p2v[rsel] == one
                    e1 = [jnp.where(m1,
                                    r1[k, pl.ds(64 + t * 16, 16)],
                                    r1[k, pl.ds(t * 16, 16)])
                          for t in range(4)]
                    e2 = [jnp.where(m2,
                                    r2[k, pl.ds(64 + t * 16, 16)],
                                    r2[k, pl.ds(t * 16, 16)])
                          for t in range(4)]
                    p = (e1[0] * e2[0] + e1[1] * e2[1]
                         + e1[2] * e2[2] + e1[3] * e2[3])
                    s = jnp.sum(p)
                    acc_s = jnp.where(lanes == r, s, acc_s)
                    for t in range(4):
                        acc1 = acc1 + e1[t] * e1[t]
                        acc2 = acc2 + e2[t] * e2[t]
                bv = biasv[pl.ds(c * _CH + gbase, 16)]
                accb = accb + bv * bv
                scores[pl.ds(c * _CH + gbase, 16)] = acc_s + bv
                return acc1, acc2, accb

            carry = lax.fori_loop(0, _CH // 16, group, carry)

        acc1, acc2, accb = carry
        sqst[pl.ds(0, 16)] = acc1
        sqst[pl.ds(16, 16)] = acc2
        sqst[pl.ds(32, 16)] = accb
        pltpu.sync_copy(scores, score_out.at[pl.ds(base, _BPW)])
        pltpu.sync_copy(sqst, sq_out.at[pl.ds(wid * 48, 48)])

    return body(nid, nbr, emd2, bias)


def _tc_loss(scores2d, label2d, sq2d):
    def body(s_ref, y_ref, q_ref, o_ref):
        s = s_ref[...]
        y = y_ref[...]
        bce = jnp.maximum(s, 0.0) - s * y + jnp.log1p(jnp.exp(-jnp.abs(s)))
        o_ref[0, 0] = jnp.sum(bce) * (1.0 / _B) + (_LAMBDA_DIS * 0.5) * jnp.sum(q_ref[...])

    return pl.pallas_call(
        body,
        out_shape=jax.ShapeDtypeStruct((1, 1), jnp.float32),
        out_specs=pl.BlockSpec(memory_space=pltpu.SMEM),
    )(scores2d, label2d, sq2d)


def kernel(node_ids, neighbor_ids, label, node_emd, bias_vector):
    emd2 = node_emd.reshape(_N_NODE // 2, 2 * _D)
    scores, sq = _sc_scores(node_ids, neighbor_ids, emd2, bias_vector)
    loss = _tc_loss(
        scores.reshape(128, 128),
        label.reshape(128, 128),
        sq.reshape(12, 128),
    )
    return loss[0, 0]


# speedup vs baseline: 1.2825x; 1.2825x over previous
import functools

import jax
import jax.numpy as jnp
from jax import lax
from jax.experimental import pallas as pl
from jax.experimental.pallas import tpu as pltpu
from jax.experimental.pallas import tpu_sc as plsc

_LAMBDA_DIS = 1e-05
_N_NODE = 100000
_B = 16384
_D = 64
_NW = 32
_BPW = _B // _NW
_NCH = 4
_CH = _BPW // _NCH


def _sc_scores(nid, nbr, emd, bias):
    mesh = plsc.VectorSubcoreMesh(core_axis_name="c", subcore_axis_name="s")

    @functools.partial(
        pl.kernel,
        out_type=(
            jax.ShapeDtypeStruct((_B,), jnp.float32),
            jax.ShapeDtypeStruct((_NW, 3, 16), jnp.float32),
        ),
        mesh=mesh,
        compiler_params=pltpu.CompilerParams(
            needs_layout_passes=False, use_tc_tiling_on_sc=False),
        scratch_types=[
            pltpu.VMEM((_NCH, _CH), jnp.int32),
            pltpu.VMEM((_NCH, _CH), jnp.int32),
            pltpu.VMEM((_BPW, _D), jnp.float32),
            pltpu.VMEM((_BPW, _D), jnp.float32),
            pltpu.VMEM((_BPW,), jnp.float32),
            pltpu.VMEM((_BPW,), jnp.float32),
            pltpu.VMEM((3, 16), jnp.float32),
            pltpu.SemaphoreType.DMA,
        ],
    )
    def body(nid_hbm, nbr_hbm, emd_hbm, bias_hbm, score_out, sq_out,
             idx1, idx2, rows1, rows2, biasv, scores, sqst, sem):
        wid = lax.axis_index("s") * 2 + lax.axis_index("c")

        pltpu.sync_copy(nid_hbm.at[wid], idx1)
        pltpu.sync_copy(nbr_hbm.at[wid], idx2)

        copies = []
        for c in range(_NCH):
            sl = pl.ds(c * _CH, _CH)
            copies.append(pltpu.async_copy(emd_hbm.at[idx1.at[c]], rows1.at[sl], sem))
            copies.append(pltpu.async_copy(emd_hbm.at[idx2.at[c]], rows2.at[sl], sem))
            copies.append(pltpu.async_copy(bias_hbm.at[idx2.at[c]], biasv.at[sl], sem))
        for cp in copies:
            cp.wait()

        lanes = lax.iota(jnp.int32, 16)
        zero = jnp.zeros((16,), jnp.float32)

        def group(g, carry):
            acc1, acc2, accb = carry
            gbase = pl.multiple_of(g * 16, 16)
            acc_s = zero
            for r in range(16):
                row = gbase + r
                e1 = [rows1[row, pl.ds(t * 16, 16)] for t in range(4)]
                e2 = [rows2[row, pl.ds(t * 16, 16)] for t in range(4)]
                p = (e1[0] * e2[0] + e1[1] * e2[1]
                     + e1[2] * e2[2] + e1[3] * e2[3])
                s = jnp.sum(p)
                acc_s = jnp.where(lanes == r, s, acc_s)
                for t in range(4):
                    acc1 = acc1 + e1[t] * e1[t]
                    acc2 = acc2 + e2[t] * e2[t]
            bv = biasv[pl.ds(gbase, 16)]
            accb = accb + bv * bv
            scores[pl.ds(gbase, 16)] = acc_s + bv
            return acc1, acc2, accb

        acc1, acc2, accb = lax.fori_loop(0, _BPW // 16, group, (zero, zero, zero))
        sqst[0, :] = acc1
        sqst[1, :] = acc2
        sqst[2, :] = accb
        pltpu.sync_copy(scores, score_out.at[pl.ds(wid * _BPW, _BPW)])
        pltpu.sync_copy(sqst, sq_out.at[wid])

    return body(nid, nbr, emd, bias)


def _tc_loss(scores2d, label2d, sq2d):
    def body(s_ref, y_ref, q_ref, o_ref):
        s = s_ref[...]
        y = y_ref[...]
        bce = jnp.maximum(s, 0.0) - s * y + jnp.log1p(jnp.exp(-jnp.abs(s)))
        o_ref[0, 0] = jnp.sum(bce) * (1.0 / _B) + (_LAMBDA_DIS * 0.5) * jnp.sum(q_ref[...])

    return pl.pallas_call(
        body,
        out_shape=jax.ShapeDtypeStruct((1, 1), jnp.float32),
        out_specs=pl.BlockSpec(memory_space=pltpu.SMEM),
    )(scores2d, label2d, sq2d)


def kernel(node_ids, neighbor_ids, label, node_emd, bias_vector):
    scores, sq = _sc_scores(
        node_ids.reshape(_NW, _NCH, _CH),
        neighbor_ids.reshape(_NW, _NCH, _CH),
        node_emd,
        bias_vector,
    )
    loss = _tc_loss(
        scores.reshape(128, 128),
        label.reshape(128, 128),
        sq.reshape(12, 128),
    )
    return loss[0, 0]
